# baseline (speedup 1.0000x reference)
import functools

import jax
import jax.numpy as jnp
from jax.experimental import pallas as pl
from jax.experimental.pallas import tpu as pltpu


def _vmem_gather_kernel(rows_ref, table_ref, out_ref, *, batch):
    for i in range(batch):
        out_ref[i, 0] = table_ref[rows_ref[i], 0]


def kernel(labels, table, force_drop_ids):
    B = labels.shape[0]
    V, H = table.shape
    num_classes = V - 1
    eff = jnp.where(force_drop_ids == 1, num_classes,
                    labels.astype(jnp.int32)).astype(jnp.int32)
    eff = jnp.clip(eff, 0, V - 1)

    if H % 384 == 0:
        HC = 384
    elif H % 128 == 0:
        HC = 128
    else:
        HC = H
    n_chunks = H // HC

    table3 = table.reshape(V, 1, H)

    out = pl.pallas_call(
        functools.partial(_vmem_gather_kernel, batch=B),
        grid=(n_chunks,),
        in_specs=[
            pl.BlockSpec(memory_space=pltpu.SMEM),
            pl.BlockSpec((V, 1, HC), lambda i: (0, 0, i)),
        ],
        out_specs=pl.BlockSpec((B, 1, HC), lambda i: (0, 0, i)),
        out_shape=jax.ShapeDtypeStruct((B, 1, H), table.dtype),
        compiler_params=pltpu.CompilerParams(
            dimension_semantics=("parallel",),
            disable_bounds_checks=True,
        ),
    )(eff, table3)
    return out.reshape(B, H)


# speedup vs baseline: 1.0075x; 1.0075x over previous
import functools

import jax
import jax.numpy as jnp
from jax.experimental import pallas as pl
from jax.experimental.pallas import tpu as pltpu


def _vmem_gather_kernel(rows_ref, table_ref, out_ref, *, block_rows):
    base = pl.program_id(0) * block_rows
    for i in range(block_rows):
        out_ref[i, 0] = table_ref[rows_ref[base + i], 0]


def kernel(labels, table, force_drop_ids):
    B = labels.shape[0]
    V, H = table.shape
    num_classes = V - 1
    eff = jnp.where(force_drop_ids == 1, num_classes,
                    labels.astype(jnp.int32)).astype(jnp.int32)
    eff = jnp.clip(eff, 0, V - 1)

    n_blocks = 2 if B % 2 == 0 else 1
    block_rows = B // n_blocks

    table3 = table.reshape(V, 1, H)

    out = pl.pallas_call(
        functools.partial(_vmem_gather_kernel, block_rows=block_rows),
        grid=(n_blocks,),
        in_specs=[
            pl.BlockSpec(memory_space=pltpu.SMEM),
            pl.BlockSpec((V, 1, H), lambda i: (0, 0, 0)),
        ],
        out_specs=pl.BlockSpec((block_rows, 1, H), lambda i: (i, 0, 0)),
        out_shape=jax.ShapeDtypeStruct((B, 1, H), table.dtype),
        compiler_params=pltpu.CompilerParams(
            dimension_semantics=("arbitrary",),
            disable_bounds_checks=True,
        ),
    )(eff, table3)
    return out.reshape(B, H)


# speedup vs baseline: 4.9196x; 4.8829x over previous
import functools

import jax
import jax.numpy as jnp
from jax.experimental import pallas as pl
from jax.experimental.pallas import tpu as pltpu


def _stream_gather_kernel(rows_ref, table_ref, out_ref, tbl3, out3,
                          sem_in, sem_out, *, batch):
    cp_in = pltpu.make_async_copy(table_ref, tbl3.at[:, 0, :], sem_in)
    cp_in.start()
    cp_in.wait()
    for i in range(batch):
        out3[i, 0] = tbl3[rows_ref[i], 0]
    cp_out = pltpu.make_async_copy(out3.at[:, 0, :], out_ref, sem_out)
    cp_out.start()
    cp_out.wait()


def kernel(labels, table, force_drop_ids):
    B = labels.shape[0]
    V, H = table.shape
    num_classes = V - 1
    eff = jnp.where(force_drop_ids == 1, num_classes,
                    labels.astype(jnp.int32)).astype(jnp.int32)
    eff = jnp.clip(eff, 0, V - 1)

    return pl.pallas_call(
        functools.partial(_stream_gather_kernel, batch=B),
        in_specs=[
            pl.BlockSpec(memory_space=pltpu.SMEM),
            pl.BlockSpec(memory_space=pltpu.HBM),
        ],
        out_specs=pl.BlockSpec(memory_space=pltpu.HBM),
        out_shape=jax.ShapeDtypeStruct((B, H), table.dtype),
        scratch_shapes=[
            pltpu.VMEM((V, 1, H), table.dtype),
            pltpu.VMEM((B, 1, H), table.dtype),
            pltpu.SemaphoreType.DMA,
            pltpu.SemaphoreType.DMA,
        ],
        compiler_params=pltpu.CompilerParams(
            disable_bounds_checks=True,
        ),
    )(eff, table)


# speedup vs baseline: 5.7538x; 1.1696x over previous
import functools

import jax
import jax.numpy as jnp
from jax.experimental import pallas as pl
from jax.experimental.pallas import tpu as pltpu


def _stream_gather_kernel(labels_ref, drop_ref, table_ref, out_ref,
                          tbl3, out3, eff, sem_in, sem_out,
                          *, batch, n_chunks):
    v, h = table_ref.shape
    vc = v // n_chunks
    tail = v - n_chunks * vc
    for c in range(n_chunks):
        pltpu.make_async_copy(
            table_ref.at[pl.ds(c * vc, vc), :],
            tbl3.at[pl.ds(c * vc, vc), 0, :],
            sem_in,
        ).start()
    if tail:
        pltpu.make_async_copy(
            table_ref.at[pl.ds(n_chunks * vc, tail), :],
            tbl3.at[pl.ds(n_chunks * vc, tail), 0, :],
            sem_in,
        ).start()

    num_classes = v - 1
    for i in range(batch):
        row = jnp.where(drop_ref[i] == 1, num_classes, labels_ref[i])
        eff[i] = jnp.clip(row, 0, num_classes)

    pltpu.make_async_copy(table_ref, tbl3.at[:, 0, :], sem_in).wait()

    half = batch // 2
    for i in range(half):
        out3[i, 0] = tbl3[eff[i], 0]
    pltpu.make_async_copy(
        out3.at[pl.ds(0, half), 0, :],
        out_ref.at[pl.ds(0, half), :],
        sem_out,
    ).start()
    for i in range(half, batch):
        out3[i, 0] = tbl3[eff[i], 0]
    pltpu.make_async_copy(
        out3.at[pl.ds(half, batch - half), 0, :],
        out_ref.at[pl.ds(half, batch - half), :],
        sem_out,
    ).start()
    pltpu.make_async_copy(out3.at[:, 0, :], out_ref, sem_out).wait()


def kernel(labels, table, force_drop_ids):
    B = labels.shape[0]
    V, H = table.shape

    return pl.pallas_call(
        functools.partial(_stream_gather_kernel, batch=B, n_chunks=16),
        in_specs=[
            pl.BlockSpec(memory_space=pltpu.SMEM),
            pl.BlockSpec(memory_space=pltpu.SMEM),
            pl.BlockSpec(memory_space=pltpu.HBM),
        ],
        out_specs=pl.BlockSpec(memory_space=pltpu.HBM),
        out_shape=jax.ShapeDtypeStruct((B, H), table.dtype),
        scratch_shapes=[
            pltpu.VMEM((V, 1, H), table.dtype),
            pltpu.VMEM((B, 1, H), table.dtype),
            pltpu.SMEM((B,), jnp.int32),
            pltpu.SemaphoreType.DMA,
            pltpu.SemaphoreType.DMA,
        ],
        compiler_params=pltpu.CompilerParams(
            disable_bounds_checks=True,
        ),
    )(labels.astype(jnp.int32), force_drop_ids.astype(jnp.int32), table)


# speedup vs baseline: 5.8903x; 1.0237x over previous
import functools

import jax
import jax.numpy as jnp
from jax.experimental import pallas as pl
from jax.experimental.pallas import tpu as pltpu


def _stream_gather_kernel(labels_ref, drop_ref, table_ref, out_ref,
                          tbl3, out3, eff, sem_in, sem_out,
                          *, batch, n_chunks):
    v, h = table_ref.shape
    vc = v // n_chunks
    tail = v - n_chunks * vc
    for c in range(n_chunks):
        pltpu.make_async_copy(
            table_ref.at[pl.ds(c * vc, vc), :],
            tbl3.at[pl.ds(c * vc, vc), 0, :],
            sem_in,
        ).start()
    if tail:
        pltpu.make_async_copy(
            table_ref.at[pl.ds(n_chunks * vc, tail), :],
            tbl3.at[pl.ds(n_chunks * vc, tail), 0, :],
            sem_in,
        ).start()

    num_classes = v - 1
    for i in range(batch):
        row = jnp.where(drop_ref[i] == 1, num_classes, labels_ref[i])
        eff[i] = jnp.clip(row, 0, num_classes)

    pltpu.make_async_copy(table_ref, tbl3.at[:, 0, :], sem_in).wait()

    q = batch // 4
    for s in range(4):
        lo = s * q
        hi = batch if s == 3 else (s + 1) * q
        for i in range(lo, hi):
            out3[i, 0] = tbl3[eff[i], 0]
        pltpu.make_async_copy(
            out3.at[pl.ds(lo, hi - lo), 0, :],
            out_ref.at[pl.ds(lo, hi - lo), :],
            sem_out,
        ).start()
    pltpu.make_async_copy(out3.at[:, 0, :], out_ref, sem_out).wait()


def kernel(labels, table, force_drop_ids):
    B = labels.shape[0]
    V, H = table.shape

    return pl.pallas_call(
        functools.partial(_stream_gather_kernel, batch=B, n_chunks=32),
        in_specs=[
            pl.BlockSpec(memory_space=pltpu.SMEM),
            pl.BlockSpec(memory_space=pltpu.SMEM),
            pl.BlockSpec(memory_space=pltpu.HBM),
        ],
        out_specs=pl.BlockSpec(memory_space=pltpu.HBM),
        out_shape=jax.ShapeDtypeStruct((B, H), table.dtype),
        scratch_shapes=[
            pltpu.VMEM((V, 1, H), table.dtype),
            pltpu.VMEM((B, 1, H), table.dtype),
            pltpu.SMEM((B,), jnp.int32),
            pltpu.SemaphoreType.DMA,
            pltpu.SemaphoreType.DMA,
        ],
        compiler_params=pltpu.CompilerParams(
            disable_bounds_checks=True,
        ),
    )(labels.astype(jnp.int32), force_drop_ids.astype(jnp.int32), table)
